# baseline (speedup 1.0000x reference)
import functools

import jax
import jax.numpy as jnp
from jax import lax
from jax.experimental import pallas as pl
from jax.experimental.pallas import tpu as pltpu
from jax.experimental.pallas import tpu_sc as plsc

_DIM = 64
_G = 16
_NBUF = 4
_RCOLS = 256


def _relayout_pairs(table_t, NC, NS):
    n_rows = table_t.shape[1]
    n_blocks = (n_rows // 128) * 128 // _RCOLS
    NW = NC * NS
    mesh = plsc.VectorSubcoreMesh(core_axis_name="c", subcore_axis_name="s")
    per_w = n_blocks // NW
    extra = n_blocks - per_w * NW

    @functools.partial(
        pl.kernel,
        mesh=mesh,
        out_type=jax.ShapeDtypeStruct((n_rows // 2, 128), jnp.float32),
        compiler_params=pltpu.CompilerParams(needs_layout_passes=False),
        scratch_types=[
            pltpu.VMEM((_DIM, _RCOLS), jnp.float32),
            pltpu.VMEM((_DIM, _RCOLS), jnp.float32),
            pltpu.VMEM((_RCOLS // 2, 128), jnp.float32),
            pltpu.VMEM((_RCOLS // 2, 128), jnp.float32),
            pltpu.SemaphoreType.DMA,
            pltpu.SemaphoreType.DMA,
            pltpu.SemaphoreType.DMA,
            pltpu.SemaphoreType.DMA,
        ],
    )
    def relayout(tt_hbm, out_hbm, in0, in1, ob0, ob1, si0, si1, so0, so1):
        wid = lax.axis_index("s") * NC + lax.axis_index("c")
        nb_w = per_w + jnp.where(wid < extra, 1, 0)
        inbufs, obufs = [in0, in1], [ob0, ob1]
        isems, osems = [si0, si1], [so0, so1]
        lane = lax.iota(jnp.int32, 16)

        def blk(t):
            return wid + t * NW

        def issue_in(t, slot):
            pltpu.async_copy(
                tt_hbm.at[:, pl.ds(blk(t) * _RCOLS, _RCOLS)],
                inbufs[slot], isems[slot])

        def wait_in(slot):
            pltpu.make_async_copy(
                tt_hbm.at[:, pl.ds(0, _RCOLS)], inbufs[slot],
                isems[slot]).wait()

        def issue_out(t, slot):
            pltpu.async_copy(
                obufs[slot],
                out_hbm.at[pl.ds(blk(t) * (_RCOLS // 2), _RCOLS // 2)],
                osems[slot])

        def wait_out(slot):
            pltpu.make_async_copy(
                obufs[slot],
                out_hbm.at[pl.ds(0, _RCOLS // 2)], osems[slot]).wait()

        def transpose(slot):
            ib, ob = inbufs[slot], obufs[slot]

            def row(k, carry):
                for c in range(8):
                    col = 2 * k + (1 if c >= 4 else 0)
                    feat = (c % 4) * 16 + lane
                    ch = plsc.load_gather(
                        ib, [feat, jnp.zeros((16,), jnp.int32) + col])
                    ob[k, pl.ds(c * 16, 16)] = ch
                return carry

            lax.fori_loop(0, _RCOLS // 2, row, 0)

        for slot in range(2):
            @pl.when(slot < nb_w)
            def _():
                issue_in(slot, slot)

        def body(i, carry):
            for slot in range(2):
                t = i * 2 + slot

                @pl.when(t < nb_w)
                def _():
                    wait_in(slot)

                    @pl.when(t >= 2)
                    def _():
                        wait_out(slot)

                    transpose(slot)
                    issue_out(t, slot)

                    @pl.when(t + 2 < nb_w)
                    def _():
                        issue_in(t + 2, slot)
            return carry

        lax.fori_loop(0, (per_w + 2) // 2, body, 0)
        for slot in range(2):
            @pl.when(nb_w >= slot + 1)
            def _():
                wait_out(slot)

    return relayout(table_t)


def kernel(user_indices, item_indices, user_table, item_table, W, b):
    B = user_indices.shape[0]
    n_rows = user_table.shape[0]
    info = plsc.get_sparse_core_info()
    NC, NS = info.num_cores, info.num_subcores
    NW = NC * NS
    b_per_w = B // NW
    n_groups = b_per_w // _G
    cut = (n_rows // 128) * 128

    ui = user_indices.astype(jnp.int32).reshape(NW, n_groups, _G)
    ii = item_indices.astype(jnp.int32).reshape(NW, n_groups, _G)
    wb = jnp.concatenate([W[:, 0], jnp.full((_G,), b[0], jnp.float32)])

    user_pairs = user_table.reshape(n_rows // 2, 128)
    item_pairs = _relayout_pairs(item_table.T, NC, NS)
    tail_i = item_table[cut:, :]

    mesh = plsc.VectorSubcoreMesh(core_axis_name="c", subcore_axis_name="s")

    @functools.partial(
        pl.kernel,
        mesh=mesh,
        out_type=jax.ShapeDtypeStruct((B,), jnp.float32),
        compiler_params=pltpu.CompilerParams(needs_layout_passes=False),
        scratch_types=[
            pltpu.VMEM((n_groups, _G), jnp.int32),
            pltpu.VMEM((n_groups, _G), jnp.int32),
            pltpu.VMEM((_NBUF * _G, 128), jnp.float32),
            pltpu.VMEM((_NBUF * _G, 128), jnp.float32),
            pltpu.VMEM((n_rows - cut, _DIM), jnp.float32),
            pltpu.VMEM((_DIM + _G,), jnp.float32),
            pltpu.VMEM((b_per_w,), jnp.float32),
            pltpu.VMEM((_G * _G,), jnp.float32),
            pltpu.SemaphoreType.DMA,
            pltpu.SemaphoreType.DMA,
            pltpu.SemaphoreType.DMA,
            pltpu.SemaphoreType.DMA,
            pltpu.SemaphoreType.DMA,
            pltpu.SemaphoreType.DMA,
            pltpu.SemaphoreType.DMA,
            pltpu.SemaphoreType.DMA,
        ],
    )
    def gmf(ui_hbm, ii_hbm, up_hbm, ip_hbm, tl_hbm, wb_hbm, out_hbm,
            idx_u, idx_i, urows, vrows, tailv, wv, out_v, tpose, *sems):
        usems, vsems = sems[:_NBUF], sems[_NBUF:]
        wid = lax.axis_index("s") * NC + lax.axis_index("c")
        base = wid * b_per_w

        pltpu.sync_copy(ui_hbm.at[wid], idx_u)
        pltpu.sync_copy(ii_hbm.at[wid], idx_i)
        pltpu.sync_copy(wb_hbm, wv)
        pltpu.sync_copy(tl_hbm, tailv)

        wc = [wv[pl.ds(c * 16, 16)] for c in range(_DIM // 16)]
        bias = wv[pl.ds(_DIM, _G)]
        lane = lax.iota(jnp.int32, 16)
        col0 = lane * 16
        max_pair = cut // 2 - 1

        def issue(g, slot):
            uvec = idx_u[g, pl.ds(0, _G)]
            ivec = jnp.minimum(idx_i[g, pl.ds(0, _G)] >> 1, max_pair)
            upair = uvec >> 1
            for j in range(_G):
                pltpu.async_copy(up_hbm.at[upair[j]],
                                 urows.at[slot * _G + j], usems[slot])
                pltpu.async_copy(ip_hbm.at[ivec[j]],
                                 vrows.at[slot * _G + j], vsems[slot])

        def drain(slot):
            for j in range(_G):
                pltpu.make_async_copy(
                    up_hbm.at[0], urows.at[slot * _G + j], usems[slot]).wait()
                pltpu.make_async_copy(
                    ip_hbm.at[0], vrows.at[slot * _G + j], vsems[slot]).wait()

        def compute(g, slot):
            uvec = idx_u[g, pl.ds(0, _G)]
            ivec = idx_i[g, pl.ds(0, _G)]
            for j in range(_G):
                ru = uvec[j]
                ri = ivec[j]
                uoff = (ru & 1) * _DIM
                ioff = (ri & 1) * _DIM
                in_tail = ri >= cut
                rt = jnp.maximum(ri, cut) - cut
                s = None
                for c in range(_DIM // 16):
                    u = urows[slot * _G + j, pl.ds(uoff + c * 16, 16)]
                    v_main = vrows[slot * _G + j, pl.ds(ioff + c * 16, 16)]
                    v_tail = tailv[rt, pl.ds(c * 16, 16)]
                    v = jnp.where(in_tail, v_tail, v_main)
                    term = u * v * wc[c]
                    s = term if s is None else s + term
                tpose[pl.ds(j * 16, 16)] = s
            acc = bias
            for j in range(_G):
                acc = acc + plsc.load_gather(tpose, [col0 + j])
            out_v[pl.ds(g * _G, _G)] = acc

        for slot in range(_NBUF):
            issue(slot, slot)

        def body(k, carry):
            for slot in range(_NBUF):
                g = k * _NBUF + slot
                drain(slot)
                compute(g, slot)

                @pl.when(g + _NBUF < n_groups)
                def _():
                    issue(g + _NBUF, slot)
            return carry

        lax.fori_loop(0, n_groups // _NBUF, body, 0)
        pltpu.sync_copy(out_v, out_hbm.at[pl.ds(base, b_per_w)])

    out = gmf(ui, ii, user_pairs, item_pairs, tail_i, wb)
    return out.reshape(B, 1)


# speedup vs baseline: 1.5685x; 1.5685x over previous
import functools

import jax
import jax.numpy as jnp
from jax import lax
from jax.experimental import pallas as pl
from jax.experimental.pallas import tpu as pltpu
from jax.experimental.pallas import tpu_sc as plsc

_DIM = 64
_G = 16
_NBUF = 4


def kernel(user_indices, item_indices, user_table, item_table, W, b):
    B = user_indices.shape[0]
    n_rows = user_table.shape[0]
    info = plsc.get_sparse_core_info()
    NC, NS = info.num_cores, info.num_subcores
    NW = NC * NS
    b_per_w = B // NW
    n_groups = b_per_w // _G

    ui = user_indices.astype(jnp.int32).reshape(NW, n_groups, _G)
    ii = item_indices.astype(jnp.int32).reshape(NW, n_groups, _G)
    wb = jnp.concatenate([W[:, 0], jnp.full((_G,), b[0], jnp.float32)])

    user_pairs = user_table.reshape(n_rows // 2, 128)
    item_pairs = item_table.reshape(n_rows // 2, 128)

    mesh = plsc.VectorSubcoreMesh(core_axis_name="c", subcore_axis_name="s")

    @functools.partial(
        pl.kernel,
        mesh=mesh,
        out_type=jax.ShapeDtypeStruct((B,), jnp.float32),
        compiler_params=pltpu.CompilerParams(needs_layout_passes=False),
        scratch_types=[
            pltpu.VMEM((n_groups, _G), jnp.int32),
            pltpu.VMEM((n_groups, _G), jnp.int32),
            pltpu.VMEM((_NBUF * _G, 128), jnp.float32),
            pltpu.VMEM((_NBUF * _G, 128), jnp.float32),
            pltpu.VMEM((_DIM + _G,), jnp.float32),
            pltpu.VMEM((b_per_w,), jnp.float32),
            pltpu.VMEM((_G * _G,), jnp.float32),
            pltpu.SemaphoreType.DMA,
            pltpu.SemaphoreType.DMA,
            pltpu.SemaphoreType.DMA,
            pltpu.SemaphoreType.DMA,
            pltpu.SemaphoreType.DMA,
            pltpu.SemaphoreType.DMA,
            pltpu.SemaphoreType.DMA,
            pltpu.SemaphoreType.DMA,
        ],
    )
    def gmf(ui_hbm, ii_hbm, up_hbm, ip_hbm, wb_hbm, out_hbm,
            idx_u, idx_i, urows, vrows, wv, out_v, tpose, *sems):
        usems, vsems = sems[:_NBUF], sems[_NBUF:]
        wid = lax.axis_index("s") * NC + lax.axis_index("c")
        base = wid * b_per_w

        pltpu.sync_copy(ui_hbm.at[wid], idx_u)
        pltpu.sync_copy(ii_hbm.at[wid], idx_i)
        pltpu.sync_copy(wb_hbm, wv)

        wc = [wv[pl.ds(c * 16, 16)] for c in range(_DIM // 16)]
        bias = wv[pl.ds(_DIM, _G)]
        lane = lax.iota(jnp.int32, 16)
        col0 = lane * 16

        def issue(g, slot):
            uvec = idx_u[g, pl.ds(0, _G)]
            ivec = idx_i[g, pl.ds(0, _G)] >> 1
            upair = uvec >> 1
            for j in range(_G):
                pltpu.async_copy(up_hbm.at[upair[j]],
                                 urows.at[slot * _G + j], usems[slot])
                pltpu.async_copy(ip_hbm.at[ivec[j]],
                                 vrows.at[slot * _G + j], vsems[slot])

        def drain(slot):
            for j in range(_G):
                pltpu.make_async_copy(
                    up_hbm.at[0], urows.at[slot * _G + j], usems[slot]).wait()
                pltpu.make_async_copy(
                    ip_hbm.at[0], vrows.at[slot * _G + j], vsems[slot]).wait()

        def compute(g, slot):
            uvec = idx_u[g, pl.ds(0, _G)]
            ivec = idx_i[g, pl.ds(0, _G)]
            for j in range(_G):
                ru = uvec[j]
                ri = ivec[j]
                uoff = (ru & 1) * _DIM
                ioff = (ri & 1) * _DIM
                s = None
                for c in range(_DIM // 16):
                    u = urows[slot * _G + j, pl.ds(uoff + c * 16, 16)]
                    v = vrows[slot * _G + j, pl.ds(ioff + c * 16, 16)]
                    term = u * v * wc[c]
                    s = term if s is None else s + term
                tpose[pl.ds(j * 16, 16)] = s
            acc = bias
            for j in range(_G):
                acc = acc + plsc.load_gather(tpose, [col0 + j])
            out_v[pl.ds(g * _G, _G)] = acc

        for slot in range(_NBUF):
            issue(slot, slot)

        def body(k, carry):
            for slot in range(_NBUF):
                g = k * _NBUF + slot
                drain(slot)
                compute(g, slot)

                @pl.when(g + _NBUF < n_groups)
                def _():
                    issue(g + _NBUF, slot)
            return carry

        lax.fori_loop(0, n_groups // _NBUF, body, 0)
        pltpu.sync_copy(out_v, out_hbm.at[pl.ds(base, b_per_w)])

    out = gmf(ui, ii, user_pairs, item_pairs, wb)
    return out.reshape(B, 1)


# speedup vs baseline: 2.4634x; 1.5706x over previous
import functools

import jax
import jax.numpy as jnp
from jax import lax
from jax.experimental import pallas as pl
from jax.experimental.pallas import tpu as pltpu
from jax.experimental.pallas import tpu_sc as plsc

_DIM = 64
_G = 16
_NBUF = 4


def kernel(user_indices, item_indices, user_table, item_table, W, b):
    B = user_indices.shape[0]
    n_rows = user_table.shape[0]
    info = plsc.get_sparse_core_info()
    NC, NS = info.num_cores, info.num_subcores
    NW = NC * NS
    b_per_w = B // NW
    n_groups = b_per_w // _G

    ui = user_indices.astype(jnp.int32).reshape(NW, n_groups, _G)
    ii = item_indices.astype(jnp.int32).reshape(NW, n_groups, _G)
    wb = jnp.concatenate([W[:, 0], jnp.full((_G,), b[0], jnp.float32)])


    mesh = plsc.VectorSubcoreMesh(core_axis_name="c", subcore_axis_name="s")

    @functools.partial(
        pl.kernel,
        mesh=mesh,
        out_type=jax.ShapeDtypeStruct((B,), jnp.float32),
        compiler_params=pltpu.CompilerParams(needs_layout_passes=False),
        scratch_types=[
            pltpu.VMEM((n_groups, _G), jnp.int32),
            pltpu.VMEM((n_groups, _G), jnp.int32),
            pltpu.VMEM((_NBUF * _G, _DIM), jnp.float32),
            pltpu.VMEM((_NBUF * _G, _DIM), jnp.float32),
            pltpu.VMEM((_DIM + _G,), jnp.float32),
            pltpu.VMEM((b_per_w,), jnp.float32),
            pltpu.VMEM((_G * _G,), jnp.float32),
            pltpu.SemaphoreType.DMA,
            pltpu.SemaphoreType.DMA,
            pltpu.SemaphoreType.DMA,
            pltpu.SemaphoreType.DMA,
            pltpu.SemaphoreType.DMA,
            pltpu.SemaphoreType.DMA,
            pltpu.SemaphoreType.DMA,
            pltpu.SemaphoreType.DMA,
        ],
    )
    def gmf(ui_hbm, ii_hbm, up_hbm, ip_hbm, wb_hbm, out_hbm,
            idx_u, idx_i, urows, vrows, wv, out_v, tpose, *sems):
        usems, vsems = sems[:_NBUF], sems[_NBUF:]
        wid = lax.axis_index("s") * NC + lax.axis_index("c")
        base = wid * b_per_w

        pltpu.sync_copy(ui_hbm.at[wid], idx_u)
        pltpu.sync_copy(ii_hbm.at[wid], idx_i)
        pltpu.sync_copy(wb_hbm, wv)

        wc = [wv[pl.ds(c * 16, 16)] for c in range(_DIM // 16)]
        bias = wv[pl.ds(_DIM, _G)]
        lane = lax.iota(jnp.int32, 16)
        col0 = lane * 16

        def issue(g, slot):
            uvec = idx_u[g, pl.ds(0, _G)]
            ivec = idx_i[g, pl.ds(0, _G)]
            for j in range(_G):
                pltpu.async_copy(up_hbm.at[uvec[j]],
                                 urows.at[slot * _G + j], usems[slot])
                pltpu.async_copy(ip_hbm.at[ivec[j]],
                                 vrows.at[slot * _G + j], vsems[slot])

        def drain(slot):
            for j in range(_G):
                pltpu.make_async_copy(
                    up_hbm.at[0], urows.at[slot * _G + j], usems[slot]).wait()
                pltpu.make_async_copy(
                    ip_hbm.at[0], vrows.at[slot * _G + j], vsems[slot]).wait()

        def compute(g, slot):
            for j in range(_G):
                s = None
                for c in range(_DIM // 16):
                    u = urows[slot * _G + j, pl.ds(c * 16, 16)]
                    v = vrows[slot * _G + j, pl.ds(c * 16, 16)]
                    term = u * v * wc[c]
                    s = term if s is None else s + term
                tpose[pl.ds(j * 16, 16)] = s
            acc = bias
            for j in range(_G):
                acc = acc + plsc.load_gather(tpose, [col0 + j])
            out_v[pl.ds(g * _G, _G)] = acc

        for slot in range(_NBUF):
            issue(slot, slot)

        def body(k, carry):
            for slot in range(_NBUF):
                g = k * _NBUF + slot
                drain(slot)
                compute(g, slot)

                @pl.when(g + _NBUF < n_groups)
                def _():
                    issue(g + _NBUF, slot)
            return carry

        lax.fori_loop(0, n_groups // _NBUF, body, 0)
        pltpu.sync_copy(out_v, out_hbm.at[pl.ds(base, b_per_w)])

    out = gmf(ui, ii, user_table, item_table, wb)
    return out.reshape(B, 1)
